# baseline (speedup 1.0000x reference)
import functools

import jax
import jax.numpy as jnp
from jax import lax
from jax.experimental import pallas as pl
from jax.experimental.pallas import tpu as pltpu
from jax.experimental.pallas import tpu_sc as plsc

N = 10000
E = 320000
D = 128
BOT = 32
NTAB = 10240
DUMMY = 10000
BLK = 5120
GRID = NTAB // BLK

NC = 2
NS = 16
NWORK = NC * NS
CH = 128
STEPS = 80
EPAD = NWORK * STEPS * CH
RPT = NTAB // NS


def _mm(a, b):
    return jax.lax.dot(a, b, precision=jax.lax.Precision.DEFAULT,
                       preferred_element_type=jnp.float32)


def _tc1_body(x_ref, fw_ref, fb_ref, w0_ref, b0_ref, w1_ref, b1_ref,
              pre1_ref, tab_ref):
    fe = jnp.maximum(_mm(x_ref[...], fw_ref[...]) + fb_ref[...], 0.0)
    pre1_ref[...] = _mm(fe, w1_ref[...][8:8 + D]) + b1_ref[...]
    tab_ref[...] = jnp.maximum(_mm(fe, w0_ref[...][8:]) + b0_ref[...], 0.0)


def _tc1(x, fenc_W, fenc_b, ae0_W, ae0_b, ae1_W, ae1_b):
    full = lambda *s: pl.BlockSpec(s, lambda i: (0,) * len(s))
    return pl.pallas_call(
        _tc1_body,
        grid=(GRID,),
        in_specs=[
            pl.BlockSpec((BLK, D), lambda i: (i, 0)),
            full(D, D), full(D), full(8 + D, BOT), full(BOT),
            full(8 + D + BOT, BOT), full(BOT),
        ],
        out_specs=[
            pl.BlockSpec((BLK, BOT), lambda i: (i, 0)),
            pl.BlockSpec((BLK, BOT), lambda i: (i, 0)),
        ],
        out_shape=[
            jax.ShapeDtypeStruct((NTAB, BOT), jnp.float32),
            jax.ShapeDtypeStruct((NTAB, BOT), jnp.float32),
        ],
    )(x, fenc_W, fenc_b, ae0_W, ae0_b, ae1_W, ae1_b)


def _sc_body(tab_hbm, src_hbm, dst_hbm, out_hbm, outc_hbm,
             src_v, dst_v, rows_v, buf_v, cbuf_v, ones_v,
             agg_s, tab_s, cnt_s, sem, sem2):
    c = lax.axis_index("c")
    s = lax.axis_index("s")
    wid = s * NC + c
    my_base = s * RPT

    pltpu.async_copy(src_hbm.at[wid], src_v, sem)
    pltpu.async_copy(dst_hbm.at[wid], dst_v, sem2)
    pltpu.sync_copy(tab_hbm.at[pl.ds(my_base, RPT)],
                    tab_s.at[pl.ds(my_base, RPT)])

    def fill(i, _):
        ones_v[pl.ds(i * 16, 16)] = jnp.ones((16,), jnp.float32)
        return 0
    lax.fori_loop(0, CH // 16, fill, 0)

    def zcrow(i, _):
        cbuf_v[pl.ds(i * 16, 16)] = jnp.zeros((16,), jnp.float32)
        return 0
    lax.fori_loop(0, RPT // 16, zcrow, 0)
    pltpu.sync_copy(cbuf_v, cnt_s.at[pl.ds(my_base, RPT)])

    def zrow(i, _):
        buf_v[i, pl.ds(0, 16)] = jnp.zeros((16,), jnp.float32)
        buf_v[i, pl.ds(16, 16)] = jnp.zeros((16,), jnp.float32)
        return 0
    lax.fori_loop(0, CH, zrow, 0)

    def zslice(k, _):
        pltpu.sync_copy(buf_v, agg_s.at[pl.ds(my_base + k * CH, CH)])
        return 0
    lax.fori_loop(0, RPT // CH, zslice, 0)

    pltpu.make_async_copy(src_hbm.at[wid], src_v, sem).wait()
    pltpu.make_async_copy(dst_hbm.at[wid], dst_v, sem2).wait()
    plsc.subcore_barrier()

    def start(jj, buf, s_):
        pltpu.async_copy(tab_s.at[src_v.at[jj]], buf, s_)

    def drain(buf, s_):
        pltpu.make_async_copy(tab_hbm.at[pl.ds(0, CH)], buf, s_).wait()

    start(0, rows_v, sem)
    start(1, buf_v, sem2)

    def step2(i, _):
        j = 2 * i

        def half(jj, buf, s_):
            drain(buf, s_)
            pltpu.sync_copy(buf, agg_s.at[dst_v.at[jj]], add=True)
            pltpu.sync_copy(ones_v, cnt_s.at[dst_v.at[jj]], add=True)

            @pl.when(jj + 2 < STEPS)
            def _():
                start(jj + 2, buf, s_)

        half(j, rows_v, sem)
        half(j + 1, buf_v, sem2)
        return 0
    lax.fori_loop(0, STEPS // 2, step2, 0)
    plsc.subcore_barrier()

    pltpu.sync_copy(agg_s.at[pl.ds(my_base, RPT)],
                    out_hbm.at[c].at[pl.ds(my_base, RPT)])
    pltpu.sync_copy(cnt_s.at[pl.ds(my_base, RPT)],
                    outc_hbm.at[c].at[pl.ds(my_base, RPT)])


def _sc_segsum(tab, srcI, dstI):
    call = functools.partial(
        pl.kernel,
        out_type=[
            jax.ShapeDtypeStruct((NC, NTAB, BOT), jnp.float32),
            jax.ShapeDtypeStruct((NC, NTAB), jnp.float32),
        ],
        mesh=plsc.VectorSubcoreMesh(
            core_axis_name="c", subcore_axis_name="s",
            num_cores=NC, num_subcores=NS),
        scratch_types=[
            pltpu.VMEM((STEPS, CH), jnp.int32),
            pltpu.VMEM((STEPS, CH), jnp.int32),
            pltpu.VMEM((CH, BOT), jnp.float32),
            pltpu.VMEM((CH, BOT), jnp.float32),
            pltpu.VMEM((RPT,), jnp.float32),
            pltpu.VMEM((CH,), jnp.float32),
            pltpu.VMEM_SHARED((NTAB, BOT), jnp.float32),
            pltpu.VMEM_SHARED((NTAB, BOT), jnp.float32),
            pltpu.VMEM_SHARED((NTAB,), jnp.float32),
            pltpu.SemaphoreType.DMA,
            pltpu.SemaphoreType.DMA,
        ],
        compiler_params=pltpu.CompilerParams(use_tc_tiling_on_sc=False),
    )(_sc_body)
    return call(tab, srcI, dstI)


def _tc2_body(pre1_ref, parts_ref, cnts_ref, sW, sb, w1_ref, dW, db,
              pW, pb, fW, fb, out_ref):
    p = parts_ref[...]
    agg = p[0] + p[1]
    cn = cnts_ref[...]
    cnt = cn[0] + cn[1]
    mean = agg / jnp.maximum(cnt, 1.0)
    summ = jnp.maximum(_mm(mean, sW[...]) + sb[...], 0.0)
    bn1 = jnp.maximum(pre1_ref[...] + _mm(summ, w1_ref[...][8 + D:]), 0.0)
    rec1 = jnp.maximum(_mm(bn1, dW[...]) + db[...], 0.0)
    proj = jnp.maximum(_mm(rec1, pW[...]) + pb[...], 0.0)
    out_ref[...] = _mm(proj, fW[...]) + fb[...]


def _tc2(pre1, parts, cnts, sW, sb, ae1_W, dW, db, pW, pb, fW, fb):
    full = lambda *s: pl.BlockSpec(s, lambda i: (0,) * len(s))
    return pl.pallas_call(
        _tc2_body,
        grid=(GRID,),
        in_specs=[
            pl.BlockSpec((BLK, BOT), lambda i: (i, 0)),
            pl.BlockSpec((NC, BLK, BOT), lambda i: (0, i, 0)),
            pl.BlockSpec((NC, BLK, 1), lambda i: (0, i, 0)),
            full(BOT, BOT), full(BOT),
            full(8 + D + BOT, BOT),
            full(BOT, 168), full(168),
            full(168, D), full(D),
            full(D, D), full(D),
        ],
        out_specs=pl.BlockSpec((BLK, D), lambda i: (i, 0)),
        out_shape=jax.ShapeDtypeStruct((N, D), jnp.float32),
    )(pre1, parts, cnts, sW, sb, ae1_W, dW, db, pW, pb, fW, fb)


def kernel(x, edge_index, fenc_W, fenc_b, ae0_enc_W, ae0_enc_b, summ_W, summ_b,
           ae1_enc_W, ae1_enc_b, ae1_dec_W, ae1_dec_b, proj_W, proj_b,
           fdec_W, fdec_b):
    pre1, tab = _tc1(x, fenc_W, fenc_b, ae0_enc_W, ae0_enc_b,
                     ae1_enc_W, ae1_enc_b)

    pad = jnp.full((EPAD - E,), DUMMY, jnp.int32)
    srcI = jnp.concatenate([edge_index[0], pad]).reshape(NWORK, STEPS, CH)
    dstI = jnp.concatenate([edge_index[1], pad]).reshape(NWORK, STEPS, CH)
    parts, cnts = _sc_segsum(tab, srcI, dstI)

    out = _tc2(pre1, parts, cnts.reshape(NC, NTAB, 1),
               summ_W, summ_b, ae1_enc_W,
               ae1_dec_W, ae1_dec_b, proj_W, proj_b, fdec_W, fdec_b)
    return out


# speedup vs baseline: 1.0249x; 1.0249x over previous
import functools

import jax
import jax.numpy as jnp
from jax import lax
from jax.experimental import pallas as pl
from jax.experimental.pallas import tpu as pltpu
from jax.experimental.pallas import tpu_sc as plsc

N = 10000
E = 320000
D = 128
BOT = 32
NTAB = 10240
DUMMY = 10000
BLK = 5120
GRID = NTAB // BLK

NC = 2
NS = 16
NWORK = NC * NS
CH = 128
STEPS = 80
EPAD = NWORK * STEPS * CH
RPT = NTAB // NS


def _mm(a, b):
    return jax.lax.dot(a, b, precision=jax.lax.Precision.DEFAULT,
                       preferred_element_type=jnp.float32)


def _tc1_body(x_ref, fw_ref, fb_ref, w0_ref, b0_ref, w1_ref, b1_ref,
              pre1_ref, tab_ref):
    fe = jnp.maximum(_mm(x_ref[...], fw_ref[...]) + fb_ref[...], 0.0)
    pre1_ref[...] = _mm(fe, w1_ref[...][8:8 + D]) + b1_ref[...]
    tab_ref[...] = jnp.maximum(_mm(fe, w0_ref[...][8:]) + b0_ref[...], 0.0)


def _tc1(x, fenc_W, fenc_b, ae0_W, ae0_b, ae1_W, ae1_b):
    full = lambda *s: pl.BlockSpec(s, lambda i: (0,) * len(s))
    return pl.pallas_call(
        _tc1_body,
        grid=(GRID,),
        in_specs=[
            pl.BlockSpec((BLK, D), lambda i: (i, 0)),
            full(D, D), full(D), full(8 + D, BOT), full(BOT),
            full(8 + D + BOT, BOT), full(BOT),
        ],
        out_specs=[
            pl.BlockSpec((BLK, BOT), lambda i: (i, 0)),
            pl.BlockSpec((BLK, BOT), lambda i: (i, 0)),
        ],
        out_shape=[
            jax.ShapeDtypeStruct((NTAB, BOT), jnp.float32),
            jax.ShapeDtypeStruct((NTAB, BOT), jnp.float32),
        ],
    )(x, fenc_W, fenc_b, ae0_W, ae0_b, ae1_W, ae1_b)


def _sc_body(tab_hbm, src_hbm, dst_hbm, out_hbm, outc_hbm,
             src_v, dst_v, rows_v, buf_v, cbuf_v, ones_v,
             agg_s, tab_s, cnt_s, sem, sem2, sem3):
    c = lax.axis_index("c")
    s = lax.axis_index("s")
    wid = s * NC + c
    my_base = s * RPT

    pltpu.async_copy(src_hbm.at[wid], src_v, sem)
    pltpu.async_copy(dst_hbm.at[wid], dst_v, sem2)
    pltpu.sync_copy(tab_hbm.at[pl.ds(my_base, RPT)],
                    tab_s.at[pl.ds(my_base, RPT)])

    def fill(i, _):
        ones_v[pl.ds(i * 16, 16)] = jnp.ones((16,), jnp.float32)
        return 0
    lax.fori_loop(0, CH // 16, fill, 0)

    def zcrow(i, _):
        cbuf_v[pl.ds(i * 16, 16)] = jnp.zeros((16,), jnp.float32)
        return 0
    lax.fori_loop(0, RPT // 16, zcrow, 0)
    pltpu.sync_copy(cbuf_v, cnt_s.at[pl.ds(my_base, RPT)])

    def zrow(i, _):
        buf_v[i, pl.ds(0, 16)] = jnp.zeros((16,), jnp.float32)
        buf_v[i, pl.ds(16, 16)] = jnp.zeros((16,), jnp.float32)
        return 0
    lax.fori_loop(0, CH, zrow, 0)

    def zslice(k, _):
        pltpu.sync_copy(buf_v, agg_s.at[pl.ds(my_base + k * CH, CH)])
        return 0
    lax.fori_loop(0, RPT // CH, zslice, 0)

    pltpu.make_async_copy(src_hbm.at[wid], src_v, sem).wait()
    pltpu.make_async_copy(dst_hbm.at[wid], dst_v, sem2).wait()
    plsc.subcore_barrier()

    def start(jj, buf, s_):
        pltpu.async_copy(tab_s.at[src_v.at[jj]], buf, s_)

    def drain(buf, s_):
        pltpu.make_async_copy(tab_hbm.at[pl.ds(0, CH)], buf, s_).wait()

    start(0, rows_v, sem)
    start(1, buf_v, sem2)

    def step2(i, _):
        j = 2 * i

        def half(jj, buf, s_):
            drain(buf, s_)
            a = pltpu.async_copy(buf, agg_s.at[dst_v.at[jj]], sem3, add=True)
            pltpu.sync_copy(ones_v, cnt_s.at[dst_v.at[jj]], add=True)
            a.wait()

            @pl.when(jj + 2 < STEPS)
            def _():
                start(jj + 2, buf, s_)

        half(j, rows_v, sem)
        half(j + 1, buf_v, sem2)
        return 0
    lax.fori_loop(0, STEPS // 2, step2, 0)
    plsc.subcore_barrier()

    pltpu.sync_copy(agg_s.at[pl.ds(my_base, RPT)],
                    out_hbm.at[c].at[pl.ds(my_base, RPT)])
    pltpu.sync_copy(cnt_s.at[pl.ds(my_base, RPT)],
                    outc_hbm.at[c].at[pl.ds(my_base, RPT)])


def _sc_segsum(tab, srcI, dstI):
    call = functools.partial(
        pl.kernel,
        out_type=[
            jax.ShapeDtypeStruct((NC, NTAB, BOT), jnp.float32),
            jax.ShapeDtypeStruct((NC, NTAB), jnp.float32),
        ],
        mesh=plsc.VectorSubcoreMesh(
            core_axis_name="c", subcore_axis_name="s",
            num_cores=NC, num_subcores=NS),
        scratch_types=[
            pltpu.VMEM((STEPS, CH), jnp.int32),
            pltpu.VMEM((STEPS, CH), jnp.int32),
            pltpu.VMEM((CH, BOT), jnp.float32),
            pltpu.VMEM((CH, BOT), jnp.float32),
            pltpu.VMEM((RPT,), jnp.float32),
            pltpu.VMEM((CH,), jnp.float32),
            pltpu.VMEM_SHARED((NTAB, BOT), jnp.float32),
            pltpu.VMEM_SHARED((NTAB, BOT), jnp.float32),
            pltpu.VMEM_SHARED((NTAB,), jnp.float32),
            pltpu.SemaphoreType.DMA,
            pltpu.SemaphoreType.DMA,
            pltpu.SemaphoreType.DMA,
        ],
        compiler_params=pltpu.CompilerParams(use_tc_tiling_on_sc=False),
    )(_sc_body)
    return call(tab, srcI, dstI)


def _tc2_body(pre1_ref, parts_ref, cnts_ref, sW, sb, w1_ref, dW, db,
              pW, pb, fW, fb, out_ref):
    p = parts_ref[...]
    agg = p[0] + p[1]
    cn = cnts_ref[...]
    cnt = cn[0] + cn[1]
    mean = agg / jnp.maximum(cnt, 1.0)
    summ = jnp.maximum(_mm(mean, sW[...]) + sb[...], 0.0)
    bn1 = jnp.maximum(pre1_ref[...] + _mm(summ, w1_ref[...][8 + D:]), 0.0)
    rec1 = jnp.maximum(_mm(bn1, dW[...]) + db[...], 0.0)
    proj = jnp.maximum(_mm(rec1, pW[...]) + pb[...], 0.0)
    out_ref[...] = _mm(proj, fW[...]) + fb[...]


def _tc2(pre1, parts, cnts, sW, sb, ae1_W, dW, db, pW, pb, fW, fb):
    full = lambda *s: pl.BlockSpec(s, lambda i: (0,) * len(s))
    return pl.pallas_call(
        _tc2_body,
        grid=(GRID,),
        in_specs=[
            pl.BlockSpec((BLK, BOT), lambda i: (i, 0)),
            pl.BlockSpec((NC, BLK, BOT), lambda i: (0, i, 0)),
            pl.BlockSpec((NC, BLK, 1), lambda i: (0, i, 0)),
            full(BOT, BOT), full(BOT),
            full(8 + D + BOT, BOT),
            full(BOT, 168), full(168),
            full(168, D), full(D),
            full(D, D), full(D),
        ],
        out_specs=pl.BlockSpec((BLK, D), lambda i: (i, 0)),
        out_shape=jax.ShapeDtypeStruct((N, D), jnp.float32),
    )(pre1, parts, cnts, sW, sb, ae1_W, dW, db, pW, pb, fW, fb)


def kernel(x, edge_index, fenc_W, fenc_b, ae0_enc_W, ae0_enc_b, summ_W, summ_b,
           ae1_enc_W, ae1_enc_b, ae1_dec_W, ae1_dec_b, proj_W, proj_b,
           fdec_W, fdec_b):
    pre1, tab = _tc1(x, fenc_W, fenc_b, ae0_enc_W, ae0_enc_b,
                     ae1_enc_W, ae1_enc_b)

    pad = jnp.full((EPAD - E,), DUMMY, jnp.int32)
    srcI = jnp.concatenate([edge_index[0], pad]).reshape(NWORK, STEPS, CH)
    dstI = jnp.concatenate([edge_index[1], pad]).reshape(NWORK, STEPS, CH)
    parts, cnts = _sc_segsum(tab, srcI, dstI)

    out = _tc2(pre1, parts, cnts.reshape(NC, NTAB, 1),
               summ_W, summ_b, ae1_enc_W,
               ae1_dec_W, ae1_dec_b, proj_W, proj_b, fdec_W, fdec_b)
    return out
